# baseline (speedup 1.0000x reference)
import functools
import math

import jax
import jax.numpy as jnp
import numpy as np
from jax import lax
from jax.experimental import pallas as pl
from jax.experimental.pallas import tpu as pltpu
from jax.experimental.pallas import tpu_sc as plsc

N = 10000
N_P = 10240
E = 320000
E_P = 327680
EB = 128
ER = E_P // EB
ROWS_PER_TILE = ER // 16
STRIPE = N_P // 16
TRASH = N_P - 1
D_FEAT = 128
D_EDGE = 16
TIME_DIM = 32
HIDDEN = 256
HALF = HIDDEN // 2
NPOW = 11
CTX_CH = 2048

_SDS = jax.ShapeDtypeStruct
_f32 = jnp.float32


def _taylor_c16():
    half = TIME_DIM // 2
    f = np.exp(-math.log(10000.0) * np.arange(half) / float(half - 1))
    f2 = np.concatenate([f, f])
    ph = np.concatenate([np.zeros(half), np.full(half, 0.5 * math.pi)])
    c = np.zeros((16, TIME_DIM))
    for j in range(NPOW):
        c[j] = (f2 ** j) * np.sin(0.5 * f2 + ph + j * 0.5 * math.pi) \
            / math.factorial(j)
    return jnp.asarray(c, dtype=_f32)


W_EDGES = E_P // 32
W_ROWS = W_EDGES // EB
NB = CTX_CH // EB


def _pow_scatter(dst2, t1d):
    mesh = plsc.VectorSubcoreMesh(core_axis_name="c", subcore_axis_name="s")

    @functools.partial(
        pl.kernel,
        out_type=[_SDS((N_P, 16), _f32), _SDS((N_P, 16), _f32)],
        mesh=mesh,
        scratch_types=[
            pltpu.VMEM((CTX_CH,), _f32),
            pltpu.VMEM((CTX_CH, 16), _f32),
            pltpu.VMEM((NB, EB), jnp.int32),
            pltpu.SemaphoreType.DMA,
            pltpu.VMEM_SHARED((N_P, 16), _f32),
        ],
        compiler_params=pltpu.CompilerParams(use_tc_tiling_on_sc=False,
                                             needs_layout_passes=False),
    )
    def k(dst_hbm, t_hbm, o0_hbm, o1_hbm, tbuf, pbuf, dstbuf, ssem, accp):
        ci = lax.axis_index("c")
        tid = lax.axis_index("s")
        w = tid * 2 + ci
        z16 = jnp.zeros((16,), _f32)
        lane = lax.iota(jnp.int32, 16)
        v0 = jnp.where(lane == 0, 1.0, 0.0).astype(_f32)

        @pl.loop(0, CTX_CH)
        def _(i):
            pbuf[i, pl.ds(0, 16)] = z16

        sl = pl.ds(tid * STRIPE, STRIPE)
        pltpu.sync_copy(pbuf.at[pl.ds(0, STRIPE)], accp.at[sl])
        plsc.subcore_barrier()

        @pl.loop(0, CTX_CH)
        def _(i):
            pbuf[i, pl.ds(0, 16)] = v0

        @pl.loop(0, W_ROWS // NB)
        def _(c):
            e0 = w * W_EDGES + c * CTX_CH
            r0 = w * W_ROWS + c * NB
            pltpu.sync_copy(dst_hbm.at[pl.ds(r0, NB)], dstbuf)
            pltpu.sync_copy(t_hbm.at[pl.ds(e0, CTX_CH)], tbuf)

            @pl.loop(0, CTX_CH // 16)
            def _(g):
                t16 = tbuf[pl.ds(g * 16, 16)]
                u = t16 - 0.5
                ridx = g * 16 + lane
                p = u
                plsc.store_scatter(
                    pbuf, [ridx, jnp.full((16,), 1, jnp.int32)], p)
                for j in range(2, NPOW):
                    p = p * u
                    plsc.store_scatter(
                        pbuf, [ridx, jnp.full((16,), j, jnp.int32)], p)

            descs = [pltpu.async_copy(pbuf.at[pl.ds(b * EB, EB)],
                                      accp.at[dstbuf.at[b]], ssem, add=True)
                     for b in range(NB)]
            for d in descs:
                d.wait()

        plsc.subcore_barrier()

        @pl.when(ci == 0)
        def _():
            pltpu.sync_copy(accp.at[sl], o0_hbm.at[sl])

        @pl.when(ci == 1)
        def _():
            pltpu.sync_copy(accp.at[sl], o1_hbm.at[sl])

    return k(dst2, t1d)


def _attr_scatter(dst2, attr):
    mesh = plsc.VectorSubcoreMesh(core_axis_name="c", subcore_axis_name="s")

    @functools.partial(
        pl.kernel,
        out_type=[_SDS((N_P, 16), _f32), _SDS((N_P, 16), _f32)],
        mesh=mesh,
        scratch_types=[
            pltpu.VMEM((CTX_CH, 16), _f32),
            pltpu.VMEM((NB, EB), jnp.int32),
            pltpu.SemaphoreType.DMA,
            pltpu.VMEM_SHARED((N_P, 16), _f32),
        ],
        compiler_params=pltpu.CompilerParams(use_tc_tiling_on_sc=False,
                                             needs_layout_passes=False),
    )
    def k(dst_hbm, a_hbm, o0_hbm, o1_hbm, astage, dstbuf, ssem, acca):
        ci = lax.axis_index("c")
        tid = lax.axis_index("s")
        w = tid * 2 + ci
        z16 = jnp.zeros((16,), _f32)

        @pl.loop(0, CTX_CH)
        def _(i):
            astage[i, pl.ds(0, 16)] = z16

        sl = pl.ds(tid * STRIPE, STRIPE)
        pltpu.sync_copy(astage.at[pl.ds(0, STRIPE)], acca.at[sl])
        plsc.subcore_barrier()

        @pl.loop(0, W_ROWS // NB)
        def _(c):
            e0 = w * W_EDGES + c * CTX_CH
            r0 = w * W_ROWS + c * NB
            pltpu.sync_copy(dst_hbm.at[pl.ds(r0, NB)], dstbuf)
            pltpu.sync_copy(a_hbm.at[pl.ds(e0, CTX_CH)], astage)
            descs = [pltpu.async_copy(astage.at[pl.ds(b * EB, EB)],
                                      acca.at[dstbuf.at[b]], ssem, add=True)
                     for b in range(NB)]
            for d in descs:
                d.wait()

        plsc.subcore_barrier()

        @pl.when(ci == 0)
        def _():
            pltpu.sync_copy(acca.at[sl], o0_hbm.at[sl])

        @pl.when(ci == 1)
        def _():
            pltpu.sync_copy(acca.at[sl], o1_hbm.at[sl])

    return k(dst2, attr)


def _conv_scatter(src2, dst2, ga, gb):
    mesh = plsc.VectorSubcoreMesh(core_axis_name="c", subcore_axis_name="s")
    CHR = 32

    @functools.partial(
        pl.kernel,
        out_type=[_SDS((N_P, HALF), _f32), _SDS((N_P, HALF), _f32)],
        mesh=mesh,
        scratch_types=[
            pltpu.VMEM((CHR, EB), jnp.int32),
            pltpu.VMEM((CHR, EB), jnp.int32),
            pltpu.VMEM((EB, HALF), _f32),
            pltpu.VMEM((EB, HALF), _f32),
            pltpu.SemaphoreType.DMA,
            pltpu.SemaphoreType.DMA,
            pltpu.SemaphoreType.DMA,
            pltpu.SemaphoreType.DMA,
            pltpu.VMEM_SHARED((N_P, HALF), _f32),
        ],
        compiler_params=pltpu.CompilerParams(use_tc_tiling_on_sc=False),
    )
    def k(src_hbm, dst_hbm, ga_hbm, gb_hbm, oa_hbm, ob_hbm,
          srcbuf, dstbuf, st0, st1, gsem0, gsem1, ssem0, ssem1, acc):
        ci = lax.axis_index("c")
        tid = lax.axis_index("s")
        z16 = jnp.zeros((16,), _f32)

        @pl.loop(0, EB)
        def _(i):
            for j in range(HALF // 16):
                st0[i, pl.ds(j * 16, 16)] = z16

        for q in range(STRIPE // EB):
            pltpu.sync_copy(st0, acc.at[pl.ds(tid * STRIPE + q * EB, EB)])
        plsc.subcore_barrier()

        stages = (st0, st1)
        gsems = (gsem0, gsem1)
        ssems = (ssem0, ssem1)

        def accumulate(g_hbm):
            @pl.loop(0, ROWS_PER_TILE // CHR)
            def _(c):
                rows = pl.ds(tid * ROWS_PER_TILE + c * CHR, CHR)
                pltpu.sync_copy(src_hbm.at[rows], srcbuf)
                pltpu.sync_copy(dst_hbm.at[rows], dstbuf)
                gd = [pltpu.async_copy(g_hbm.at[srcbuf.at[0]], st0, gsem0),
                      None]
                sd = [None, None]
                for j in range(CHR):
                    b = j % 2
                    gd[b].wait()
                    sd_new = pltpu.async_copy(stages[b],
                                              acc.at[dstbuf.at[j]],
                                              ssems[b], add=True)
                    if j + 1 < CHR:
                        if sd[(j + 1) % 2] is not None:
                            sd[(j + 1) % 2].wait()
                        gd[(j + 1) % 2] = pltpu.async_copy(
                            g_hbm.at[srcbuf.at[j + 1]],
                            stages[(j + 1) % 2], gsems[(j + 1) % 2])
                    sd[b] = sd_new
                sd[0].wait()
                sd[1].wait()

        @pl.when(ci == 0)
        def _():
            accumulate(ga_hbm)

        @pl.when(ci == 1)
        def _():
            accumulate(gb_hbm)

        plsc.subcore_barrier()

        sl = pl.ds(tid * STRIPE, STRIPE)

        @pl.when(ci == 0)
        def _():
            pltpu.sync_copy(acc.at[sl], oa_hbm.at[sl])

        @pl.when(ci == 1)
        def _():
            pltpu.sync_copy(acc.at[sl], ob_hbm.at[sl])

    return k(src2, dst2, ga, gb)


_R = 1280


def _dinv_from_p(p_blk):
    counts = p_blk[:, 0:1]
    return lax.rsqrt(counts + 1.0)


def _input_body(x_ref, p0_ref, p1_ref, a0_ref, a1_ref, c_ref,
                wx_ref, wt_ref, we_ref, b_ref, w1_ref,
                ga_ref, gb_ref, p_ref):
    p = p0_ref[...] + p1_ref[...]
    p_ref[...] = p
    counts = p[:, 0:1]
    denom = jnp.maximum(counts, 1.0)
    ntc = jnp.dot(p, c_ref[...], preferred_element_type=_f32) / denom
    nec = (a0_ref[...] + a1_ref[...]) / denom
    h = jnp.dot(x_ref[...], wx_ref[...], preferred_element_type=_f32)
    h += jnp.dot(ntc, wt_ref[...], preferred_element_type=_f32)
    h += jnp.dot(nec, we_ref[...], preferred_element_type=_f32)
    h = jnp.maximum(h + b_ref[...], 0.0)
    g = jnp.dot(h, w1_ref[...], preferred_element_type=_f32)
    g = g * _dinv_from_p(p)
    ga_ref[...] = g[:, :HALF]
    gb_ref[...] = g[:, HALF:]


def _input_layer(x_pad, p0, p1, a0, a1, c16, W_x, W_t, W_e, b_in2, W1):
    return pl.pallas_call(
        _input_body,
        grid=(N_P // _R,),
        in_specs=[
            pl.BlockSpec((_R, D_FEAT), lambda i: (i, 0)),
            pl.BlockSpec((_R, 16), lambda i: (i, 0)),
            pl.BlockSpec((_R, 16), lambda i: (i, 0)),
            pl.BlockSpec((_R, 16), lambda i: (i, 0)),
            pl.BlockSpec((_R, 16), lambda i: (i, 0)),
            pl.BlockSpec((16, TIME_DIM), lambda i: (0, 0)),
            pl.BlockSpec((D_FEAT, HIDDEN), lambda i: (0, 0)),
            pl.BlockSpec((TIME_DIM, HIDDEN), lambda i: (0, 0)),
            pl.BlockSpec((D_EDGE, HIDDEN), lambda i: (0, 0)),
            pl.BlockSpec((1, HIDDEN), lambda i: (0, 0)),
            pl.BlockSpec((HIDDEN, HIDDEN), lambda i: (0, 0)),
        ],
        out_specs=[
            pl.BlockSpec((_R, HALF), lambda i: (i, 0)),
            pl.BlockSpec((_R, HALF), lambda i: (i, 0)),
            pl.BlockSpec((_R, 16), lambda i: (i, 0)),
        ],
        out_shape=[_SDS((N_P, HALF), _f32), _SDS((N_P, HALF), _f32),
                   _SDS((N_P, 16), _f32)],
    )(x_pad, p0, p1, a0, a1, c16, W_x, W_t, W_e, b_in2, W1)


def _mid_body(aa_ref, ab_ref, ga_ref, gb_ref, p_ref, b_ref, w_ref,
              oa_ref, ob_ref):
    dinv = _dinv_from_p(p_ref[...])
    m = jnp.concatenate([aa_ref[...] + ga_ref[...],
                         ab_ref[...] + gb_ref[...]], axis=1)
    h = jnp.maximum(dinv * m + b_ref[...], 0.0)
    g = jnp.dot(h, w_ref[...], preferred_element_type=_f32) * dinv
    oa_ref[...] = g[:, :HALF]
    ob_ref[...] = g[:, HALF:]


def _mid_layer(aa, ab, ga, gb, accp, b2d, W):
    return pl.pallas_call(
        _mid_body,
        grid=(N_P // _R,),
        in_specs=[
            pl.BlockSpec((_R, HALF), lambda i: (i, 0)),
            pl.BlockSpec((_R, HALF), lambda i: (i, 0)),
            pl.BlockSpec((_R, HALF), lambda i: (i, 0)),
            pl.BlockSpec((_R, HALF), lambda i: (i, 0)),
            pl.BlockSpec((_R, 16), lambda i: (i, 0)),
            pl.BlockSpec((1, HIDDEN), lambda i: (0, 0)),
            pl.BlockSpec((HIDDEN, HIDDEN), lambda i: (0, 0)),
        ],
        out_specs=[
            pl.BlockSpec((_R, HALF), lambda i: (i, 0)),
            pl.BlockSpec((_R, HALF), lambda i: (i, 0)),
        ],
        out_shape=[_SDS((N_P, HALF), _f32), _SDS((N_P, HALF), _f32)],
    )(aa, ab, ga, gb, accp, b2d, W)


def _out_body(aa_ref, ab_ref, ga_ref, gb_ref, p_ref, b_ref, wc_ref, bc_ref,
              o_ref):
    dinv = _dinv_from_p(p_ref[...])
    m = jnp.concatenate([aa_ref[...] + ga_ref[...],
                         ab_ref[...] + gb_ref[...]], axis=1)
    h = jnp.maximum(dinv * m + b_ref[...], 0.0)
    o_ref[...] = (jnp.dot(h, wc_ref[...], preferred_element_type=_f32)
                  + bc_ref[...])


def _out_layer(aa, ab, ga, gb, accp, b2d, W_cls, b_cls2):
    return pl.pallas_call(
        _out_body,
        grid=(N_P // _R,),
        in_specs=[
            pl.BlockSpec((_R, HALF), lambda i: (i, 0)),
            pl.BlockSpec((_R, HALF), lambda i: (i, 0)),
            pl.BlockSpec((_R, HALF), lambda i: (i, 0)),
            pl.BlockSpec((_R, HALF), lambda i: (i, 0)),
            pl.BlockSpec((_R, 16), lambda i: (i, 0)),
            pl.BlockSpec((1, HIDDEN), lambda i: (0, 0)),
            pl.BlockSpec((HIDDEN, 1), lambda i: (0, 0)),
            pl.BlockSpec((1, 1), lambda i: (0, 0)),
        ],
        out_specs=pl.BlockSpec((_R, 1), lambda i: (i, 0)),
        out_shape=_SDS((N_P, 1), _f32),
    )(aa, ab, ga, gb, accp, b2d, W_cls, b_cls2)


def kernel(x, edge_index, edge_time, edge_attr,
           W_in, b_in, W1, b1, W2, b2, W_cls, b_cls):
    pad_e = E_P - E
    src2 = jnp.concatenate(
        [edge_index[0], jnp.zeros((pad_e,), jnp.int32)]).reshape(ER, EB)
    dst2 = jnp.concatenate(
        [edge_index[1], jnp.full((pad_e,), TRASH, jnp.int32)]).reshape(ER, EB)
    t_p = jnp.pad(edge_time, (0, pad_e))
    attr_p = jnp.pad(edge_attr, ((0, pad_e), (0, 0)))
    c16 = _taylor_c16()

    p0, p1 = _pow_scatter(dst2, t_p)
    a0, a1 = _attr_scatter(dst2, attr_p)

    x_pad = jnp.pad(x, ((0, N_P - N), (0, 0)))
    ga, gb, accp = _input_layer(x_pad, p0, p1, a0, a1, c16,
                                W_in[:D_FEAT],
                                W_in[D_FEAT:D_FEAT + TIME_DIM],
                                W_in[D_FEAT + TIME_DIM:],
                                b_in.reshape(1, HIDDEN), W1)
    aa, ab = _conv_scatter(src2, dst2, ga, gb)
    g2a, g2b = _mid_layer(aa, ab, ga, gb, accp, b1.reshape(1, HIDDEN), W2)
    aa2, ab2 = _conv_scatter(src2, dst2, g2a, g2b)
    out = _out_layer(aa2, ab2, g2a, g2b, accp, b2.reshape(1, HIDDEN),
                     W_cls, b_cls.reshape(1, 1))
    return out[:N, 0]


# speedup vs baseline: 1.0036x; 1.0036x over previous
import functools
import math

import jax
import jax.numpy as jnp
import numpy as np
from jax import lax
from jax.experimental import pallas as pl
from jax.experimental.pallas import tpu as pltpu
from jax.experimental.pallas import tpu_sc as plsc

N = 10000
N_P = 10240
E = 320000
E_P = 327680
EB = 128
ER = E_P // EB
ROWS_PER_TILE = ER // 16
STRIPE = N_P // 16
TRASH = N_P - 1
D_FEAT = 128
D_EDGE = 16
TIME_DIM = 32
HIDDEN = 256
HALF = HIDDEN // 2
NPOW = 11
CTX_CH = 2048

_SDS = jax.ShapeDtypeStruct
_f32 = jnp.float32


def _taylor_c16():
    half = TIME_DIM // 2
    f = np.exp(-math.log(10000.0) * np.arange(half) / float(half - 1))
    f2 = np.concatenate([f, f])
    ph = np.concatenate([np.zeros(half), np.full(half, 0.5 * math.pi)])
    c = np.zeros((16, TIME_DIM))
    for j in range(NPOW):
        c[j] = (f2 ** j) * np.sin(0.5 * f2 + ph + j * 0.5 * math.pi) \
            / math.factorial(j)
    return jnp.asarray(c, dtype=_f32)


W_EDGES = E_P // 32
W_ROWS = W_EDGES // EB
NB = CTX_CH // EB


def _pow_scatter(dst2, t1d):
    mesh = plsc.VectorSubcoreMesh(core_axis_name="c", subcore_axis_name="s")

    @functools.partial(
        pl.kernel,
        out_type=[_SDS((N_P, 16), _f32), _SDS((N_P, 16), _f32)],
        mesh=mesh,
        scratch_types=[
            pltpu.VMEM((CTX_CH,), _f32),
            pltpu.VMEM((CTX_CH, 16), _f32),
            pltpu.VMEM((NB, EB), jnp.int32),
            pltpu.SemaphoreType.DMA,
            pltpu.VMEM_SHARED((N_P, 16), _f32),
        ],
        compiler_params=pltpu.CompilerParams(use_tc_tiling_on_sc=False,
                                             needs_layout_passes=False),
    )
    def k(dst_hbm, t_hbm, o0_hbm, o1_hbm, tbuf, pbuf, dstbuf, ssem, accp):
        ci = lax.axis_index("c")
        tid = lax.axis_index("s")
        w = tid * 2 + ci
        z16 = jnp.zeros((16,), _f32)
        lane = lax.iota(jnp.int32, 16)
        v0 = jnp.where(lane == 0, 1.0, 0.0).astype(_f32)

        @pl.loop(0, CTX_CH)
        def _(i):
            pbuf[i, pl.ds(0, 16)] = z16

        sl = pl.ds(tid * STRIPE, STRIPE)
        pltpu.sync_copy(pbuf.at[pl.ds(0, STRIPE)], accp.at[sl])
        plsc.subcore_barrier()

        @pl.loop(0, CTX_CH)
        def _(i):
            pbuf[i, pl.ds(0, 16)] = v0

        @pl.loop(0, W_ROWS // NB)
        def _(c):
            e0 = w * W_EDGES + c * CTX_CH
            r0 = w * W_ROWS + c * NB
            pltpu.sync_copy(dst_hbm.at[pl.ds(r0, NB)], dstbuf)
            pltpu.sync_copy(t_hbm.at[pl.ds(e0, CTX_CH)], tbuf)

            @pl.loop(0, CTX_CH // 16)
            def _(g):
                t16 = tbuf[pl.ds(g * 16, 16)]
                u = t16 - 0.5
                ridx = g * 16 + lane
                p = u
                plsc.store_scatter(
                    pbuf, [ridx, jnp.full((16,), 1, jnp.int32)], p)
                for j in range(2, NPOW):
                    p = p * u
                    plsc.store_scatter(
                        pbuf, [ridx, jnp.full((16,), j, jnp.int32)], p)

            descs = [pltpu.async_copy(pbuf.at[pl.ds(b * EB, EB)],
                                      accp.at[dstbuf.at[b]], ssem, add=True)
                     for b in range(NB)]
            for d in descs:
                d.wait()

        plsc.subcore_barrier()

        @pl.when(ci == 0)
        def _():
            pltpu.sync_copy(accp.at[sl], o0_hbm.at[sl])

        @pl.when(ci == 1)
        def _():
            pltpu.sync_copy(accp.at[sl], o1_hbm.at[sl])

    return k(dst2, t1d)


def _attr_scatter(dst2, attr):
    mesh = plsc.VectorSubcoreMesh(core_axis_name="c", subcore_axis_name="s")

    @functools.partial(
        pl.kernel,
        out_type=[_SDS((N_P, 16), _f32), _SDS((N_P, 16), _f32)],
        mesh=mesh,
        scratch_types=[
            pltpu.VMEM((CTX_CH, 16), _f32),
            pltpu.VMEM((NB, EB), jnp.int32),
            pltpu.SemaphoreType.DMA,
            pltpu.VMEM_SHARED((N_P, 16), _f32),
        ],
        compiler_params=pltpu.CompilerParams(use_tc_tiling_on_sc=False,
                                             needs_layout_passes=False),
    )
    def k(dst_hbm, a_hbm, o0_hbm, o1_hbm, astage, dstbuf, ssem, acca):
        ci = lax.axis_index("c")
        tid = lax.axis_index("s")
        w = tid * 2 + ci
        z16 = jnp.zeros((16,), _f32)

        @pl.loop(0, CTX_CH)
        def _(i):
            astage[i, pl.ds(0, 16)] = z16

        sl = pl.ds(tid * STRIPE, STRIPE)
        pltpu.sync_copy(astage.at[pl.ds(0, STRIPE)], acca.at[sl])
        plsc.subcore_barrier()

        @pl.loop(0, W_ROWS // NB)
        def _(c):
            e0 = w * W_EDGES + c * CTX_CH
            r0 = w * W_ROWS + c * NB
            pltpu.sync_copy(dst_hbm.at[pl.ds(r0, NB)], dstbuf)
            pltpu.sync_copy(a_hbm.at[pl.ds(e0, CTX_CH)], astage)
            descs = [pltpu.async_copy(astage.at[pl.ds(b * EB, EB)],
                                      acca.at[dstbuf.at[b]], ssem, add=True)
                     for b in range(NB)]
            for d in descs:
                d.wait()

        plsc.subcore_barrier()

        @pl.when(ci == 0)
        def _():
            pltpu.sync_copy(acca.at[sl], o0_hbm.at[sl])

        @pl.when(ci == 1)
        def _():
            pltpu.sync_copy(acca.at[sl], o1_hbm.at[sl])

    return k(dst2, attr)


def _conv_scatter(src2, dst2, ga, gb):
    mesh = plsc.VectorSubcoreMesh(core_axis_name="c", subcore_axis_name="s")
    CHR = 32

    @functools.partial(
        pl.kernel,
        out_type=[_SDS((N_P, HALF), _f32), _SDS((N_P, HALF), _f32)],
        mesh=mesh,
        scratch_types=[
            pltpu.VMEM((CHR, EB), jnp.int32),
            pltpu.VMEM((CHR, EB), jnp.int32),
            pltpu.VMEM((EB, HALF), _f32),
            pltpu.VMEM((EB, HALF), _f32),
            pltpu.SemaphoreType.DMA,
            pltpu.SemaphoreType.DMA,
            pltpu.SemaphoreType.DMA,
            pltpu.SemaphoreType.DMA,
            pltpu.VMEM_SHARED((N_P, HALF), _f32),
        ],
        compiler_params=pltpu.CompilerParams(use_tc_tiling_on_sc=False),
    )
    def k(src_hbm, dst_hbm, ga_hbm, gb_hbm, oa_hbm, ob_hbm,
          srcbuf, dstbuf, st0, st1, gsem0, gsem1, ssem0, ssem1, acc):
        ci = lax.axis_index("c")
        tid = lax.axis_index("s")
        z16 = jnp.zeros((16,), _f32)

        @pl.loop(0, EB)
        def _(i):
            for j in range(HALF // 16):
                st0[i, pl.ds(j * 16, 16)] = z16

        for q in range(STRIPE // EB):
            pltpu.sync_copy(st0, acc.at[pl.ds(tid * STRIPE + q * EB, EB)])
        plsc.subcore_barrier()

        stages = (st0, st1)
        gsems = (gsem0, gsem1)
        ssems = (ssem0, ssem1)

        def accumulate(g_hbm):
            @pl.loop(0, ROWS_PER_TILE // CHR)
            def _(c):
                rows = pl.ds(tid * ROWS_PER_TILE + c * CHR, CHR)
                pltpu.sync_copy(src_hbm.at[rows], srcbuf)
                pltpu.sync_copy(dst_hbm.at[rows], dstbuf)
                gd = [pltpu.async_copy(g_hbm.at[srcbuf.at[0]], st0, gsem0),
                      None]
                sd = [None, None]
                for j in range(CHR):
                    b = j % 2
                    gd[b].wait()
                    sd_new = pltpu.async_copy(stages[b],
                                              acc.at[dstbuf.at[j]],
                                              ssems[b], add=True)
                    if j + 1 < CHR:
                        if sd[(j + 1) % 2] is not None:
                            sd[(j + 1) % 2].wait()
                        gd[(j + 1) % 2] = pltpu.async_copy(
                            g_hbm.at[srcbuf.at[j + 1]],
                            stages[(j + 1) % 2], gsems[(j + 1) % 2])
                    sd[b] = sd_new
                sd[0].wait()
                sd[1].wait()

        @pl.when(ci == 0)
        def _():
            accumulate(ga_hbm)

        @pl.when(ci == 1)
        def _():
            accumulate(gb_hbm)

        plsc.subcore_barrier()

        sl = pl.ds(tid * STRIPE, STRIPE)

        @pl.when(ci == 0)
        def _():
            pltpu.sync_copy(acc.at[sl], oa_hbm.at[sl])

        @pl.when(ci == 1)
        def _():
            pltpu.sync_copy(acc.at[sl], ob_hbm.at[sl])

    return k(src2, dst2, ga, gb)


_R = 1280


def _dinv_from_p(p_blk):
    counts = p_blk[:, 0:1]
    return lax.rsqrt(counts + 1.0)


def _input_body(x_ref, p0_ref, p1_ref, a0_ref, a1_ref, c_ref,
                wx_ref, wt_ref, we_ref, b_ref, w1_ref,
                ga_ref, gb_ref, p_ref):
    p = p0_ref[...] + p1_ref[...]
    p_ref[...] = p
    counts = p[:, 0:1]
    denom = jnp.maximum(counts, 1.0)
    ntc = jnp.dot(p, c_ref[...], preferred_element_type=_f32) / denom
    nec = (a0_ref[...] + a1_ref[...]) / denom
    h = jnp.dot(x_ref[...], wx_ref[...], preferred_element_type=_f32)
    h += jnp.dot(ntc, wt_ref[...], preferred_element_type=_f32)
    h += jnp.dot(nec, we_ref[...], preferred_element_type=_f32)
    h = jnp.maximum(h + b_ref[...], 0.0)
    g = jnp.dot(h, w1_ref[...], preferred_element_type=_f32)
    g = g * _dinv_from_p(p)
    ga_ref[...] = g[:, :HALF]
    gb_ref[...] = g[:, HALF:]


def _input_layer(x_pad, p0, p1, a0, a1, c16, W_x, W_t, W_e, b_in2, W1):
    return pl.pallas_call(
        _input_body,
        grid=(N_P // _R,),
        in_specs=[
            pl.BlockSpec((_R, D_FEAT), lambda i: (i, 0)),
            pl.BlockSpec((_R, 16), lambda i: (i, 0)),
            pl.BlockSpec((_R, 16), lambda i: (i, 0)),
            pl.BlockSpec((_R, 16), lambda i: (i, 0)),
            pl.BlockSpec((_R, 16), lambda i: (i, 0)),
            pl.BlockSpec((16, TIME_DIM), lambda i: (0, 0)),
            pl.BlockSpec((D_FEAT, HIDDEN), lambda i: (0, 0)),
            pl.BlockSpec((TIME_DIM, HIDDEN), lambda i: (0, 0)),
            pl.BlockSpec((D_EDGE, HIDDEN), lambda i: (0, 0)),
            pl.BlockSpec((1, HIDDEN), lambda i: (0, 0)),
            pl.BlockSpec((HIDDEN, HIDDEN), lambda i: (0, 0)),
        ],
        out_specs=[
            pl.BlockSpec((_R, HALF), lambda i: (i, 0)),
            pl.BlockSpec((_R, HALF), lambda i: (i, 0)),
            pl.BlockSpec((_R, 16), lambda i: (i, 0)),
        ],
        out_shape=[_SDS((N_P, HALF), _f32), _SDS((N_P, HALF), _f32),
                   _SDS((N_P, 16), _f32)],
    )(x_pad, p0, p1, a0, a1, c16, W_x, W_t, W_e, b_in2, W1)


def _mid_body(aa_ref, ab_ref, ga_ref, gb_ref, p_ref, b_ref, w_ref,
              oa_ref, ob_ref):
    dinv = _dinv_from_p(p_ref[...])
    m = jnp.concatenate([aa_ref[...] + ga_ref[...],
                         ab_ref[...] + gb_ref[...]], axis=1)
    h = jnp.maximum(dinv * m + b_ref[...], 0.0)
    g = jnp.dot(h, w_ref[...], preferred_element_type=_f32) * dinv
    oa_ref[...] = g[:, :HALF]
    ob_ref[...] = g[:, HALF:]


def _mid_layer(aa, ab, ga, gb, accp, b2d, W):
    return pl.pallas_call(
        _mid_body,
        grid=(N_P // _R,),
        in_specs=[
            pl.BlockSpec((_R, HALF), lambda i: (i, 0)),
            pl.BlockSpec((_R, HALF), lambda i: (i, 0)),
            pl.BlockSpec((_R, HALF), lambda i: (i, 0)),
            pl.BlockSpec((_R, HALF), lambda i: (i, 0)),
            pl.BlockSpec((_R, 16), lambda i: (i, 0)),
            pl.BlockSpec((1, HIDDEN), lambda i: (0, 0)),
            pl.BlockSpec((HIDDEN, HIDDEN), lambda i: (0, 0)),
        ],
        out_specs=[
            pl.BlockSpec((_R, HALF), lambda i: (i, 0)),
            pl.BlockSpec((_R, HALF), lambda i: (i, 0)),
        ],
        out_shape=[_SDS((N_P, HALF), _f32), _SDS((N_P, HALF), _f32)],
    )(aa, ab, ga, gb, accp, b2d, W)


def _out_body(aa_ref, ab_ref, ga_ref, gb_ref, p_ref, b_ref, wc_ref, bc_ref,
              o_ref):
    dinv = _dinv_from_p(p_ref[...])
    m = jnp.concatenate([aa_ref[...] + ga_ref[...],
                         ab_ref[...] + gb_ref[...]], axis=1)
    h = jnp.maximum(dinv * m + b_ref[...], 0.0)
    o_ref[...] = (jnp.dot(h, wc_ref[...], preferred_element_type=_f32)
                  + bc_ref[...])


def _out_layer(aa, ab, ga, gb, accp, b2d, W_cls, b_cls2):
    return pl.pallas_call(
        _out_body,
        grid=(N_P // _R,),
        in_specs=[
            pl.BlockSpec((_R, HALF), lambda i: (i, 0)),
            pl.BlockSpec((_R, HALF), lambda i: (i, 0)),
            pl.BlockSpec((_R, HALF), lambda i: (i, 0)),
            pl.BlockSpec((_R, HALF), lambda i: (i, 0)),
            pl.BlockSpec((_R, 16), lambda i: (i, 0)),
            pl.BlockSpec((1, HIDDEN), lambda i: (0, 0)),
            pl.BlockSpec((HIDDEN, 1), lambda i: (0, 0)),
            pl.BlockSpec((1, 1), lambda i: (0, 0)),
        ],
        out_specs=pl.BlockSpec((_R, 1), lambda i: (i, 0)),
        out_shape=_SDS((N_P, 1), _f32),
    )(aa, ab, ga, gb, accp, b2d, W_cls, b_cls2)


def kernel(x, edge_index, edge_time, edge_attr,
           W_in, b_in, W1, b1, W2, b2, W_cls, b_cls):
    pad_e = E_P - E
    src2 = jnp.concatenate(
        [edge_index[0], jnp.zeros((pad_e,), jnp.int32)]).reshape(ER, EB)
    trash = N + jnp.arange(pad_e, dtype=jnp.int32) % (N_P - N)
    dst2 = jnp.concatenate([edge_index[1], trash]).reshape(ER, EB)
    t_p = jnp.pad(edge_time, (0, pad_e))
    attr_p = jnp.pad(edge_attr, ((0, pad_e), (0, 0)))
    c16 = _taylor_c16()

    p0, p1 = _pow_scatter(dst2, t_p)
    a0, a1 = _attr_scatter(dst2, attr_p)

    x_pad = jnp.pad(x, ((0, N_P - N), (0, 0)))
    ga, gb, accp = _input_layer(x_pad, p0, p1, a0, a1, c16,
                                W_in[:D_FEAT],
                                W_in[D_FEAT:D_FEAT + TIME_DIM],
                                W_in[D_FEAT + TIME_DIM:],
                                b_in.reshape(1, HIDDEN), W1)
    aa, ab = _conv_scatter(src2, dst2, ga, gb)
    g2a, g2b = _mid_layer(aa, ab, ga, gb, accp, b1.reshape(1, HIDDEN), W2)
    aa2, ab2 = _conv_scatter(src2, dst2, g2a, g2b)
    out = _out_layer(aa2, ab2, g2a, g2b, accp, b2.reshape(1, HIDDEN),
                     W_cls, b_cls.reshape(1, 1))
    return out[:N, 0]


# speedup vs baseline: 2.1114x; 2.1038x over previous
import functools
import math

import jax
import jax.numpy as jnp
import numpy as np
from jax import lax
from jax.experimental import pallas as pl
from jax.experimental.pallas import tpu as pltpu
from jax.experimental.pallas import tpu_sc as plsc

N = 10000
N_P = 10240
E = 320000
EB = 125
ER = E // EB
ROWS_PER_TILE = ER // 16
STRIPE = N_P // 16
D_FEAT = 128
D_EDGE = 16
TIME_DIM = 32
HIDDEN = 256
HALF = HIDDEN // 2
NPOW = 11
CTX_CH = 2000

_SDS = jax.ShapeDtypeStruct
_f32 = jnp.float32


def _taylor_c16():
    half = TIME_DIM // 2
    f = np.exp(-math.log(10000.0) * np.arange(half) / float(half - 1))
    f2 = np.concatenate([f, f])
    ph = np.concatenate([np.zeros(half), np.full(half, 0.5 * math.pi)])
    c = np.zeros((16, TIME_DIM))
    for j in range(NPOW):
        c[j] = (f2 ** j) * np.sin(0.5 * f2 + ph + j * 0.5 * math.pi) \
            / math.factorial(j)
    return jnp.asarray(c, dtype=_f32)


W_EDGES = E // 32
W_ROWS = W_EDGES // EB
NB = CTX_CH // EB


def _pow_scatter(dst2, t1d):
    mesh = plsc.VectorSubcoreMesh(core_axis_name="c", subcore_axis_name="s")

    @functools.partial(
        pl.kernel,
        out_type=[_SDS((N_P, 16), _f32), _SDS((N_P, 16), _f32)],
        mesh=mesh,
        scratch_types=[
            pltpu.VMEM((CTX_CH,), _f32),
            pltpu.VMEM((CTX_CH, 16), _f32),
            pltpu.VMEM((NB, EB), jnp.int32),
            pltpu.SemaphoreType.DMA,
            pltpu.VMEM_SHARED((N_P, 16), _f32),
        ],
        compiler_params=pltpu.CompilerParams(use_tc_tiling_on_sc=False,
                                             needs_layout_passes=False),
    )
    def k(dst_hbm, t_hbm, o0_hbm, o1_hbm, tbuf, pbuf, dstbuf, ssem, accp):
        ci = lax.axis_index("c")
        tid = lax.axis_index("s")
        w = tid * 2 + ci
        z16 = jnp.zeros((16,), _f32)
        lane = lax.iota(jnp.int32, 16)
        v0 = jnp.where(lane == 0, 1.0, 0.0).astype(_f32)

        @pl.loop(0, CTX_CH)
        def _(i):
            pbuf[i, pl.ds(0, 16)] = z16

        sl = pl.ds(tid * STRIPE, STRIPE)
        pltpu.sync_copy(pbuf.at[pl.ds(0, STRIPE)], accp.at[sl])
        plsc.subcore_barrier()

        @pl.loop(0, CTX_CH)
        def _(i):
            pbuf[i, pl.ds(0, 16)] = v0

        @pl.loop(0, W_ROWS // NB)
        def _(c):
            e0 = w * W_EDGES + c * CTX_CH
            r0 = w * W_ROWS + c * NB
            pltpu.sync_copy(dst_hbm.at[pl.ds(r0, NB)], dstbuf)
            pltpu.sync_copy(t_hbm.at[pl.ds(e0, CTX_CH)], tbuf)

            @pl.loop(0, CTX_CH // 16)
            def _(g):
                t16 = tbuf[pl.ds(g * 16, 16)]
                u = t16 - 0.5
                ridx = g * 16 + lane
                p = u
                plsc.store_scatter(
                    pbuf, [ridx, jnp.full((16,), 1, jnp.int32)], p)
                for j in range(2, NPOW):
                    p = p * u
                    plsc.store_scatter(
                        pbuf, [ridx, jnp.full((16,), j, jnp.int32)], p)

            descs = [pltpu.async_copy(pbuf.at[pl.ds(b * EB, EB)],
                                      accp.at[dstbuf.at[b]], ssem, add=True)
                     for b in range(NB)]
            for d in descs:
                d.wait()

        plsc.subcore_barrier()

        @pl.when(ci == 0)
        def _():
            pltpu.sync_copy(accp.at[sl], o0_hbm.at[sl])

        @pl.when(ci == 1)
        def _():
            pltpu.sync_copy(accp.at[sl], o1_hbm.at[sl])

    return k(dst2, t1d)


def _attr_scatter(dst2, attr):
    mesh = plsc.VectorSubcoreMesh(core_axis_name="c", subcore_axis_name="s")

    @functools.partial(
        pl.kernel,
        out_type=[_SDS((N_P, 16), _f32), _SDS((N_P, 16), _f32)],
        mesh=mesh,
        scratch_types=[
            pltpu.VMEM((CTX_CH, 16), _f32),
            pltpu.VMEM((NB, EB), jnp.int32),
            pltpu.SemaphoreType.DMA,
            pltpu.VMEM_SHARED((N_P, 16), _f32),
        ],
        compiler_params=pltpu.CompilerParams(use_tc_tiling_on_sc=False,
                                             needs_layout_passes=False),
    )
    def k(dst_hbm, a_hbm, o0_hbm, o1_hbm, astage, dstbuf, ssem, acca):
        ci = lax.axis_index("c")
        tid = lax.axis_index("s")
        w = tid * 2 + ci
        z16 = jnp.zeros((16,), _f32)

        @pl.loop(0, CTX_CH)
        def _(i):
            astage[i, pl.ds(0, 16)] = z16

        sl = pl.ds(tid * STRIPE, STRIPE)
        pltpu.sync_copy(astage.at[pl.ds(0, STRIPE)], acca.at[sl])
        plsc.subcore_barrier()

        @pl.loop(0, W_ROWS // NB)
        def _(c):
            e0 = w * W_EDGES + c * CTX_CH
            r0 = w * W_ROWS + c * NB
            pltpu.sync_copy(dst_hbm.at[pl.ds(r0, NB)], dstbuf)
            pltpu.sync_copy(a_hbm.at[pl.ds(e0, CTX_CH)], astage)
            descs = [pltpu.async_copy(astage.at[pl.ds(b * EB, EB)],
                                      acca.at[dstbuf.at[b]], ssem, add=True)
                     for b in range(NB)]
            for d in descs:
                d.wait()

        plsc.subcore_barrier()

        @pl.when(ci == 0)
        def _():
            pltpu.sync_copy(acca.at[sl], o0_hbm.at[sl])

        @pl.when(ci == 1)
        def _():
            pltpu.sync_copy(acca.at[sl], o1_hbm.at[sl])

    return k(dst2, attr)


def _conv_scatter(src2, dst2, ga, gb):
    mesh = plsc.VectorSubcoreMesh(core_axis_name="c", subcore_axis_name="s")
    CHR = 32

    @functools.partial(
        pl.kernel,
        out_type=[_SDS((N_P, HALF), _f32), _SDS((N_P, HALF), _f32)],
        mesh=mesh,
        scratch_types=[
            pltpu.VMEM((CHR, EB), jnp.int32),
            pltpu.VMEM((CHR, EB), jnp.int32),
            pltpu.VMEM((EB, HALF), _f32),
            pltpu.VMEM((EB, HALF), _f32),
            pltpu.SemaphoreType.DMA,
            pltpu.SemaphoreType.DMA,
            pltpu.SemaphoreType.DMA,
            pltpu.SemaphoreType.DMA,
            pltpu.VMEM_SHARED((N_P, HALF), _f32),
        ],
        compiler_params=pltpu.CompilerParams(use_tc_tiling_on_sc=False),
    )
    def k(src_hbm, dst_hbm, ga_hbm, gb_hbm, oa_hbm, ob_hbm,
          srcbuf, dstbuf, st0, st1, gsem0, gsem1, ssem0, ssem1, acc):
        ci = lax.axis_index("c")
        tid = lax.axis_index("s")
        z16 = jnp.zeros((16,), _f32)

        @pl.loop(0, EB)
        def _(i):
            for j in range(HALF // 16):
                st0[i, pl.ds(j * 16, 16)] = z16

        for q in range(5):
            pltpu.sync_copy(st0, acc.at[pl.ds(tid * STRIPE + q * EB, EB)])
        pltpu.sync_copy(st0.at[pl.ds(0, STRIPE - 5 * EB)],
                        acc.at[pl.ds(tid * STRIPE + 5 * EB, STRIPE - 5 * EB)])
        plsc.subcore_barrier()

        stages = (st0, st1)
        gsems = (gsem0, gsem1)
        ssems = (ssem0, ssem1)

        def accumulate(g_hbm):
            @pl.loop(0, ROWS_PER_TILE // CHR)
            def _(c):
                rows = pl.ds(tid * ROWS_PER_TILE + c * CHR, CHR)
                pltpu.sync_copy(src_hbm.at[rows], srcbuf)
                pltpu.sync_copy(dst_hbm.at[rows], dstbuf)
                gd = [pltpu.async_copy(g_hbm.at[srcbuf.at[0]], st0, gsem0),
                      None]
                sd = [None, None]
                for j in range(CHR):
                    b = j % 2
                    gd[b].wait()
                    sd_new = pltpu.async_copy(stages[b],
                                              acc.at[dstbuf.at[j]],
                                              ssems[b], add=True)
                    if j + 1 < CHR:
                        if sd[(j + 1) % 2] is not None:
                            sd[(j + 1) % 2].wait()
                        gd[(j + 1) % 2] = pltpu.async_copy(
                            g_hbm.at[srcbuf.at[j + 1]],
                            stages[(j + 1) % 2], gsems[(j + 1) % 2])
                    sd[b] = sd_new
                sd[0].wait()
                sd[1].wait()

        @pl.when(ci == 0)
        def _():
            accumulate(ga_hbm)

        @pl.when(ci == 1)
        def _():
            accumulate(gb_hbm)

        plsc.subcore_barrier()

        sl = pl.ds(tid * STRIPE, STRIPE)

        @pl.when(ci == 0)
        def _():
            pltpu.sync_copy(acc.at[sl], oa_hbm.at[sl])

        @pl.when(ci == 1)
        def _():
            pltpu.sync_copy(acc.at[sl], ob_hbm.at[sl])

    return k(src2, dst2, ga, gb)


_R = 1280


def _dinv_from_p(p_blk):
    counts = p_blk[:, 0:1]
    return lax.rsqrt(counts + 1.0)


def _input_body(x_ref, p0_ref, p1_ref, a0_ref, a1_ref, c_ref,
                wx_ref, wt_ref, we_ref, b_ref, w1_ref,
                ga_ref, gb_ref, p_ref):
    p = p0_ref[...] + p1_ref[...]
    p_ref[...] = p
    counts = p[:, 0:1]
    denom = jnp.maximum(counts, 1.0)
    ntc = jnp.dot(p, c_ref[...], preferred_element_type=_f32) / denom
    nec = (a0_ref[...] + a1_ref[...]) / denom
    h = jnp.dot(x_ref[...], wx_ref[...], preferred_element_type=_f32)
    h += jnp.dot(ntc, wt_ref[...], preferred_element_type=_f32)
    h += jnp.dot(nec, we_ref[...], preferred_element_type=_f32)
    h = jnp.maximum(h + b_ref[...], 0.0)
    g = jnp.dot(h, w1_ref[...], preferred_element_type=_f32)
    g = g * _dinv_from_p(p)
    ga_ref[...] = g[:, :HALF]
    gb_ref[...] = g[:, HALF:]


def _input_layer(x_pad, p0, p1, a0, a1, c16, W_x, W_t, W_e, b_in2, W1):
    return pl.pallas_call(
        _input_body,
        grid=(N_P // _R,),
        in_specs=[
            pl.BlockSpec((_R, D_FEAT), lambda i: (i, 0)),
            pl.BlockSpec((_R, 16), lambda i: (i, 0)),
            pl.BlockSpec((_R, 16), lambda i: (i, 0)),
            pl.BlockSpec((_R, 16), lambda i: (i, 0)),
            pl.BlockSpec((_R, 16), lambda i: (i, 0)),
            pl.BlockSpec((16, TIME_DIM), lambda i: (0, 0)),
            pl.BlockSpec((D_FEAT, HIDDEN), lambda i: (0, 0)),
            pl.BlockSpec((TIME_DIM, HIDDEN), lambda i: (0, 0)),
            pl.BlockSpec((D_EDGE, HIDDEN), lambda i: (0, 0)),
            pl.BlockSpec((1, HIDDEN), lambda i: (0, 0)),
            pl.BlockSpec((HIDDEN, HIDDEN), lambda i: (0, 0)),
        ],
        out_specs=[
            pl.BlockSpec((_R, HALF), lambda i: (i, 0)),
            pl.BlockSpec((_R, HALF), lambda i: (i, 0)),
            pl.BlockSpec((_R, 16), lambda i: (i, 0)),
        ],
        out_shape=[_SDS((N_P, HALF), _f32), _SDS((N_P, HALF), _f32),
                   _SDS((N_P, 16), _f32)],
    )(x_pad, p0, p1, a0, a1, c16, W_x, W_t, W_e, b_in2, W1)


def _mid_body(aa_ref, ab_ref, ga_ref, gb_ref, p_ref, b_ref, w_ref,
              oa_ref, ob_ref):
    dinv = _dinv_from_p(p_ref[...])
    m = jnp.concatenate([aa_ref[...] + ga_ref[...],
                         ab_ref[...] + gb_ref[...]], axis=1)
    h = jnp.maximum(dinv * m + b_ref[...], 0.0)
    g = jnp.dot(h, w_ref[...], preferred_element_type=_f32) * dinv
    oa_ref[...] = g[:, :HALF]
    ob_ref[...] = g[:, HALF:]


def _mid_layer(aa, ab, ga, gb, accp, b2d, W):
    return pl.pallas_call(
        _mid_body,
        grid=(N_P // _R,),
        in_specs=[
            pl.BlockSpec((_R, HALF), lambda i: (i, 0)),
            pl.BlockSpec((_R, HALF), lambda i: (i, 0)),
            pl.BlockSpec((_R, HALF), lambda i: (i, 0)),
            pl.BlockSpec((_R, HALF), lambda i: (i, 0)),
            pl.BlockSpec((_R, 16), lambda i: (i, 0)),
            pl.BlockSpec((1, HIDDEN), lambda i: (0, 0)),
            pl.BlockSpec((HIDDEN, HIDDEN), lambda i: (0, 0)),
        ],
        out_specs=[
            pl.BlockSpec((_R, HALF), lambda i: (i, 0)),
            pl.BlockSpec((_R, HALF), lambda i: (i, 0)),
        ],
        out_shape=[_SDS((N_P, HALF), _f32), _SDS((N_P, HALF), _f32)],
    )(aa, ab, ga, gb, accp, b2d, W)


def _out_body(aa_ref, ab_ref, ga_ref, gb_ref, p_ref, b_ref, wc_ref, bc_ref,
              o_ref):
    dinv = _dinv_from_p(p_ref[...])
    m = jnp.concatenate([aa_ref[...] + ga_ref[...],
                         ab_ref[...] + gb_ref[...]], axis=1)
    h = jnp.maximum(dinv * m + b_ref[...], 0.0)
    o_ref[...] = (jnp.dot(h, wc_ref[...], preferred_element_type=_f32)
                  + bc_ref[...])


def _out_layer(aa, ab, ga, gb, accp, b2d, W_cls, b_cls2):
    return pl.pallas_call(
        _out_body,
        grid=(N_P // _R,),
        in_specs=[
            pl.BlockSpec((_R, HALF), lambda i: (i, 0)),
            pl.BlockSpec((_R, HALF), lambda i: (i, 0)),
            pl.BlockSpec((_R, HALF), lambda i: (i, 0)),
            pl.BlockSpec((_R, HALF), lambda i: (i, 0)),
            pl.BlockSpec((_R, 16), lambda i: (i, 0)),
            pl.BlockSpec((1, HIDDEN), lambda i: (0, 0)),
            pl.BlockSpec((HIDDEN, 1), lambda i: (0, 0)),
            pl.BlockSpec((1, 1), lambda i: (0, 0)),
        ],
        out_specs=pl.BlockSpec((_R, 1), lambda i: (i, 0)),
        out_shape=_SDS((N_P, 1), _f32),
    )(aa, ab, ga, gb, accp, b2d, W_cls, b_cls2)


def kernel(x, edge_index, edge_time, edge_attr,
           W_in, b_in, W1, b1, W2, b2, W_cls, b_cls):
    src2 = edge_index[0].reshape(ER, EB)
    dst2 = edge_index[1].reshape(ER, EB)
    c16 = _taylor_c16()

    p0, p1 = _pow_scatter(dst2, edge_time)
    a0, a1 = _attr_scatter(dst2, edge_attr)

    x_pad = jnp.pad(x, ((0, N_P - N), (0, 0)))
    ga, gb, accp = _input_layer(x_pad, p0, p1, a0, a1, c16,
                                W_in[:D_FEAT],
                                W_in[D_FEAT:D_FEAT + TIME_DIM],
                                W_in[D_FEAT + TIME_DIM:],
                                b_in.reshape(1, HIDDEN), W1)
    aa, ab = _conv_scatter(src2, dst2, ga, gb)
    g2a, g2b = _mid_layer(aa, ab, ga, gb, accp, b1.reshape(1, HIDDEN), W2)
    aa2, ab2 = _conv_scatter(src2, dst2, g2a, g2b)
    out = _out_layer(aa2, ab2, g2a, g2b, accp, b2.reshape(1, HIDDEN),
                     W_cls, b_cls.reshape(1, 1))
    return out[:N, 0]


# speedup vs baseline: 2.1140x; 1.0012x over previous
import functools
import math

import jax
import jax.numpy as jnp
import numpy as np
from jax import lax
from jax.experimental import pallas as pl
from jax.experimental.pallas import tpu as pltpu
from jax.experimental.pallas import tpu_sc as plsc

N = 10000
N_P = 10240
E = 320000
EB = 125
ER = E // EB
ROWS_PER_TILE = ER // 16
STRIPE = N_P // 16
D_FEAT = 128
D_EDGE = 16
TIME_DIM = 32
HIDDEN = 256
HALF = HIDDEN // 2
NPOW = 11
CTX_CH = 2000

_SDS = jax.ShapeDtypeStruct
_f32 = jnp.float32


def _taylor_c16():
    half = TIME_DIM // 2
    f = np.exp(-math.log(10000.0) * np.arange(half) / float(half - 1))
    f2 = np.concatenate([f, f])
    ph = np.concatenate([np.zeros(half), np.full(half, 0.5 * math.pi)])
    c = np.zeros((16, TIME_DIM))
    for j in range(NPOW):
        c[j] = (f2 ** j) * np.sin(0.5 * f2 + ph + j * 0.5 * math.pi) \
            / math.factorial(j)
    return jnp.asarray(c, dtype=_f32)


W_EDGES = E // 32
W_ROWS = W_EDGES // EB
NB = CTX_CH // EB


def _pow_scatter(dst2, t1d):
    mesh = plsc.VectorSubcoreMesh(core_axis_name="c", subcore_axis_name="s")

    @functools.partial(
        pl.kernel,
        out_type=[_SDS((N_P, 16), _f32), _SDS((N_P, 16), _f32)],
        mesh=mesh,
        scratch_types=[
            pltpu.VMEM((CTX_CH,), _f32),
            pltpu.VMEM((CTX_CH, 16), _f32),
            pltpu.VMEM((NB, EB), jnp.int32),
            pltpu.SemaphoreType.DMA,
            pltpu.VMEM_SHARED((N_P, 16), _f32),
        ],
        compiler_params=pltpu.CompilerParams(use_tc_tiling_on_sc=False,
                                             needs_layout_passes=False),
    )
    def k(dst_hbm, t_hbm, o0_hbm, o1_hbm, tbuf, pbuf, dstbuf, ssem, accp):
        ci = lax.axis_index("c")
        tid = lax.axis_index("s")
        w = tid * 2 + ci
        z16 = jnp.zeros((16,), _f32)
        lane = lax.iota(jnp.int32, 16)
        v0 = jnp.where(lane == 0, 1.0, 0.0).astype(_f32)

        @pl.loop(0, CTX_CH)
        def _(i):
            pbuf[i, pl.ds(0, 16)] = z16

        sl = pl.ds(tid * STRIPE, STRIPE)
        pltpu.sync_copy(pbuf.at[pl.ds(0, STRIPE)], accp.at[sl])
        plsc.subcore_barrier()

        @pl.loop(0, CTX_CH)
        def _(i):
            pbuf[i, pl.ds(0, 16)] = v0

        @pl.loop(0, W_ROWS // NB)
        def _(c):
            e0 = w * W_EDGES + c * CTX_CH
            r0 = w * W_ROWS + c * NB
            pltpu.sync_copy(dst_hbm.at[pl.ds(r0, NB)], dstbuf)
            pltpu.sync_copy(t_hbm.at[pl.ds(e0, CTX_CH)], tbuf)

            @pl.loop(0, CTX_CH // 16)
            def _(g):
                t16 = tbuf[pl.ds(g * 16, 16)]
                u = t16 - 0.5
                ridx = g * 16 + lane
                p = u
                plsc.store_scatter(
                    pbuf, [ridx, jnp.full((16,), 1, jnp.int32)], p)
                for j in range(2, NPOW):
                    p = p * u
                    plsc.store_scatter(
                        pbuf, [ridx, jnp.full((16,), j, jnp.int32)], p)

            descs = [pltpu.async_copy(pbuf.at[pl.ds(b * EB, EB)],
                                      accp.at[dstbuf.at[b]], ssem, add=True)
                     for b in range(NB)]
            for d in descs:
                d.wait()

        plsc.subcore_barrier()

        @pl.when(ci == 0)
        def _():
            pltpu.sync_copy(accp.at[sl], o0_hbm.at[sl])

        @pl.when(ci == 1)
        def _():
            pltpu.sync_copy(accp.at[sl], o1_hbm.at[sl])

    return k(dst2, t1d)


def _attr_scatter(dst2, attr):
    mesh = plsc.VectorSubcoreMesh(core_axis_name="c", subcore_axis_name="s")

    @functools.partial(
        pl.kernel,
        out_type=[_SDS((N_P, 16), _f32), _SDS((N_P, 16), _f32)],
        mesh=mesh,
        scratch_types=[
            pltpu.VMEM((CTX_CH, 16), _f32),
            pltpu.VMEM((NB, EB), jnp.int32),
            pltpu.SemaphoreType.DMA,
            pltpu.VMEM_SHARED((N_P, 16), _f32),
        ],
        compiler_params=pltpu.CompilerParams(use_tc_tiling_on_sc=False,
                                             needs_layout_passes=False),
    )
    def k(dst_hbm, a_hbm, o0_hbm, o1_hbm, astage, dstbuf, ssem, acca):
        ci = lax.axis_index("c")
        tid = lax.axis_index("s")
        w = tid * 2 + ci
        z16 = jnp.zeros((16,), _f32)

        @pl.loop(0, CTX_CH)
        def _(i):
            astage[i, pl.ds(0, 16)] = z16

        sl = pl.ds(tid * STRIPE, STRIPE)
        pltpu.sync_copy(astage.at[pl.ds(0, STRIPE)], acca.at[sl])
        plsc.subcore_barrier()

        @pl.loop(0, W_ROWS // NB)
        def _(c):
            e0 = w * W_EDGES + c * CTX_CH
            r0 = w * W_ROWS + c * NB
            pltpu.sync_copy(dst_hbm.at[pl.ds(r0, NB)], dstbuf)
            pltpu.sync_copy(a_hbm.at[pl.ds(e0, CTX_CH)], astage)
            descs = [pltpu.async_copy(astage.at[pl.ds(b * EB, EB)],
                                      acca.at[dstbuf.at[b]], ssem, add=True)
                     for b in range(NB)]
            for d in descs:
                d.wait()

        plsc.subcore_barrier()

        @pl.when(ci == 0)
        def _():
            pltpu.sync_copy(acca.at[sl], o0_hbm.at[sl])

        @pl.when(ci == 1)
        def _():
            pltpu.sync_copy(acca.at[sl], o1_hbm.at[sl])

    return k(dst2, attr)


def _conv_scatter(src2, dst2, ga, gb):
    mesh = plsc.VectorSubcoreMesh(core_axis_name="c", subcore_axis_name="s")
    CHR = 32

    @functools.partial(
        pl.kernel,
        out_type=[_SDS((N_P, HALF), _f32), _SDS((N_P, HALF), _f32)],
        mesh=mesh,
        scratch_types=[
            pltpu.VMEM((CHR, EB), jnp.int32),
            pltpu.VMEM((CHR, EB), jnp.int32),
            pltpu.VMEM((EB, HALF), _f32),
            pltpu.VMEM((EB, HALF), _f32),
            pltpu.SemaphoreType.DMA,
            pltpu.SemaphoreType.DMA,
            pltpu.SemaphoreType.DMA,
            pltpu.SemaphoreType.DMA,
            pltpu.VMEM_SHARED((N_P, HALF), _f32),
        ],
        compiler_params=pltpu.CompilerParams(use_tc_tiling_on_sc=False),
    )
    def k(src_hbm, dst_hbm, ga_hbm, gb_hbm, oa_hbm, ob_hbm,
          srcbuf, dstbuf, st0, st1, gsem0, gsem1, ssem0, ssem1, acc):
        ci = lax.axis_index("c")
        tid = lax.axis_index("s")
        z16 = jnp.zeros((16,), _f32)

        @pl.loop(0, EB)
        def _(i):
            for j in range(HALF // 16):
                st0[i, pl.ds(j * 16, 16)] = z16

        for q in range(5):
            pltpu.sync_copy(st0, acc.at[pl.ds(tid * STRIPE + q * EB, EB)])
        pltpu.sync_copy(st0.at[pl.ds(0, STRIPE - 5 * EB)],
                        acc.at[pl.ds(tid * STRIPE + 5 * EB, STRIPE - 5 * EB)])
        plsc.subcore_barrier()

        stages = (st0, st1)
        gsems = (gsem0, gsem1)
        ssems = (ssem0, ssem1)

        def accumulate(g_hbm):
            @pl.loop(0, ROWS_PER_TILE // CHR)
            def _(c):
                rows = pl.ds(tid * ROWS_PER_TILE + c * CHR, CHR)
                pltpu.sync_copy(src_hbm.at[rows], srcbuf)
                pltpu.sync_copy(dst_hbm.at[rows], dstbuf)
                gd = [pltpu.async_copy(g_hbm.at[srcbuf.at[0]], st0, gsem0),
                      None]
                sd = [None, None]
                for j in range(CHR):
                    b = j % 2
                    gd[b].wait()
                    sd_new = pltpu.async_copy(stages[b],
                                              acc.at[dstbuf.at[j]],
                                              ssems[b], add=True)
                    if j + 1 < CHR:
                        if sd[(j + 1) % 2] is not None:
                            sd[(j + 1) % 2].wait()
                        gd[(j + 1) % 2] = pltpu.async_copy(
                            g_hbm.at[srcbuf.at[j + 1]],
                            stages[(j + 1) % 2], gsems[(j + 1) % 2])
                    sd[b] = sd_new
                sd[0].wait()
                sd[1].wait()

        @pl.when(ci == 0)
        def _():
            accumulate(ga_hbm)

        @pl.when(ci == 1)
        def _():
            accumulate(gb_hbm)

        plsc.subcore_barrier()

        sl = pl.ds(tid * STRIPE, STRIPE)

        @pl.when(ci == 0)
        def _():
            pltpu.sync_copy(acc.at[sl], oa_hbm.at[sl])

        @pl.when(ci == 1)
        def _():
            pltpu.sync_copy(acc.at[sl], ob_hbm.at[sl])

    return k(src2, dst2, ga, gb)


_R = 1280


def _dinv_from_p(p_blk):
    counts = p_blk[:, 0:1]
    return lax.rsqrt(counts + 1.0)


def _bdot(a, b):
    return jnp.dot(a.astype(jnp.bfloat16), b.astype(jnp.bfloat16),
                   preferred_element_type=_f32)


def _input_body(x_ref, p0_ref, p1_ref, a0_ref, a1_ref, c_ref,
                wx_ref, wt_ref, we_ref, b_ref, w1_ref,
                ga_ref, gb_ref, p_ref):
    p = p0_ref[...] + p1_ref[...]
    p_ref[...] = p
    counts = p[:, 0:1]
    denom = jnp.maximum(counts, 1.0)
    ntc = jnp.dot(p, c_ref[...], preferred_element_type=_f32) / denom
    nec = (a0_ref[...] + a1_ref[...]) / denom
    h = _bdot(x_ref[...], wx_ref[...])
    h += jnp.dot(ntc, wt_ref[...], preferred_element_type=_f32)
    h += jnp.dot(nec, we_ref[...], preferred_element_type=_f32)
    h = jnp.maximum(h + b_ref[...], 0.0)
    g = _bdot(h, w1_ref[...])
    g = g * _dinv_from_p(p)
    ga_ref[...] = g[:, :HALF]
    gb_ref[...] = g[:, HALF:]


def _input_layer(x_pad, p0, p1, a0, a1, c16, W_x, W_t, W_e, b_in2, W1):
    return pl.pallas_call(
        _input_body,
        grid=(N_P // _R,),
        in_specs=[
            pl.BlockSpec((_R, D_FEAT), lambda i: (i, 0)),
            pl.BlockSpec((_R, 16), lambda i: (i, 0)),
            pl.BlockSpec((_R, 16), lambda i: (i, 0)),
            pl.BlockSpec((_R, 16), lambda i: (i, 0)),
            pl.BlockSpec((_R, 16), lambda i: (i, 0)),
            pl.BlockSpec((16, TIME_DIM), lambda i: (0, 0)),
            pl.BlockSpec((D_FEAT, HIDDEN), lambda i: (0, 0)),
            pl.BlockSpec((TIME_DIM, HIDDEN), lambda i: (0, 0)),
            pl.BlockSpec((D_EDGE, HIDDEN), lambda i: (0, 0)),
            pl.BlockSpec((1, HIDDEN), lambda i: (0, 0)),
            pl.BlockSpec((HIDDEN, HIDDEN), lambda i: (0, 0)),
        ],
        out_specs=[
            pl.BlockSpec((_R, HALF), lambda i: (i, 0)),
            pl.BlockSpec((_R, HALF), lambda i: (i, 0)),
            pl.BlockSpec((_R, 16), lambda i: (i, 0)),
        ],
        out_shape=[_SDS((N_P, HALF), _f32), _SDS((N_P, HALF), _f32),
                   _SDS((N_P, 16), _f32)],
    )(x_pad, p0, p1, a0, a1, c16, W_x, W_t, W_e, b_in2, W1)


def _mid_body(aa_ref, ab_ref, ga_ref, gb_ref, p_ref, b_ref, w_ref,
              oa_ref, ob_ref):
    dinv = _dinv_from_p(p_ref[...])
    m = jnp.concatenate([aa_ref[...] + ga_ref[...],
                         ab_ref[...] + gb_ref[...]], axis=1)
    h = jnp.maximum(dinv * m + b_ref[...], 0.0)
    g = _bdot(h, w_ref[...]) * dinv
    oa_ref[...] = g[:, :HALF]
    ob_ref[...] = g[:, HALF:]


def _mid_layer(aa, ab, ga, gb, accp, b2d, W):
    return pl.pallas_call(
        _mid_body,
        grid=(N_P // _R,),
        in_specs=[
            pl.BlockSpec((_R, HALF), lambda i: (i, 0)),
            pl.BlockSpec((_R, HALF), lambda i: (i, 0)),
            pl.BlockSpec((_R, HALF), lambda i: (i, 0)),
            pl.BlockSpec((_R, HALF), lambda i: (i, 0)),
            pl.BlockSpec((_R, 16), lambda i: (i, 0)),
            pl.BlockSpec((1, HIDDEN), lambda i: (0, 0)),
            pl.BlockSpec((HIDDEN, HIDDEN), lambda i: (0, 0)),
        ],
        out_specs=[
            pl.BlockSpec((_R, HALF), lambda i: (i, 0)),
            pl.BlockSpec((_R, HALF), lambda i: (i, 0)),
        ],
        out_shape=[_SDS((N_P, HALF), _f32), _SDS((N_P, HALF), _f32)],
    )(aa, ab, ga, gb, accp, b2d, W)


def _out_body(aa_ref, ab_ref, ga_ref, gb_ref, p_ref, b_ref, wc_ref, bc_ref,
              o_ref):
    dinv = _dinv_from_p(p_ref[...])
    m = jnp.concatenate([aa_ref[...] + ga_ref[...],
                         ab_ref[...] + gb_ref[...]], axis=1)
    h = jnp.maximum(dinv * m + b_ref[...], 0.0)
    o_ref[...] = (jnp.dot(h, wc_ref[...], preferred_element_type=_f32)
                  + bc_ref[...])


def _out_layer(aa, ab, ga, gb, accp, b2d, W_cls, b_cls2):
    return pl.pallas_call(
        _out_body,
        grid=(N_P // _R,),
        in_specs=[
            pl.BlockSpec((_R, HALF), lambda i: (i, 0)),
            pl.BlockSpec((_R, HALF), lambda i: (i, 0)),
            pl.BlockSpec((_R, HALF), lambda i: (i, 0)),
            pl.BlockSpec((_R, HALF), lambda i: (i, 0)),
            pl.BlockSpec((_R, 16), lambda i: (i, 0)),
            pl.BlockSpec((1, HIDDEN), lambda i: (0, 0)),
            pl.BlockSpec((HIDDEN, 1), lambda i: (0, 0)),
            pl.BlockSpec((1, 1), lambda i: (0, 0)),
        ],
        out_specs=pl.BlockSpec((_R, 1), lambda i: (i, 0)),
        out_shape=_SDS((N_P, 1), _f32),
    )(aa, ab, ga, gb, accp, b2d, W_cls, b_cls2)


def kernel(x, edge_index, edge_time, edge_attr,
           W_in, b_in, W1, b1, W2, b2, W_cls, b_cls):
    src2 = edge_index[0].reshape(ER, EB)
    dst2 = edge_index[1].reshape(ER, EB)
    c16 = _taylor_c16()

    p0, p1 = _pow_scatter(dst2, edge_time)
    a0, a1 = _attr_scatter(dst2, edge_attr)

    x_pad = jnp.pad(x, ((0, N_P - N), (0, 0)))
    ga, gb, accp = _input_layer(x_pad, p0, p1, a0, a1, c16,
                                W_in[:D_FEAT],
                                W_in[D_FEAT:D_FEAT + TIME_DIM],
                                W_in[D_FEAT + TIME_DIM:],
                                b_in.reshape(1, HIDDEN), W1)
    aa, ab = _conv_scatter(src2, dst2, ga, gb)
    g2a, g2b = _mid_layer(aa, ab, ga, gb, accp, b1.reshape(1, HIDDEN), W2)
    aa2, ab2 = _conv_scatter(src2, dst2, g2a, g2b)
    out = _out_layer(aa2, ab2, g2a, g2b, accp, b2.reshape(1, HIDDEN),
                     W_cls, b_cls.reshape(1, 1))
    return out[:N, 0]
